# Initial kernel scaffold; baseline (speedup 1.0000x reference)
import jax
import jax.numpy as jnp
from jax.experimental import pallas as pl


def kernel(node_feat, edge_index, W, a, conv1_w, conv1_b, conv2_w, conv2_b):
    raise NotImplementedError("write your pallas kernel here")


# speedup vs baseline: 14.7804x; 14.7804x over previous
import functools

import jax
import jax.numpy as jnp
from jax import lax
from jax.experimental import pallas as pl
from jax.experimental.pallas import tpu as pltpu
from jax.experimental.pallas import tpu_sc as plsc

N = 10000
E = 320000
D = 128
L = 32
H = 4
HL = H * L
K = 30
C1 = 16
C2 = 32
KW2 = 5

NPAD = 10240
WIDTH = 144
NC, NS = 2, 16
NW = NC * NS
EW = E // NW
CH = 80
NCHUNK = EW // CH
ROWS_PER_TILE = NPAD // NS


def _proj_body(x_ref, wext_ref, wsrc_ref, hext_ref, asrc_ref):
    x = x_ref[...]
    hext_ref[...] = jnp.dot(x, wext_ref[...], preferred_element_type=jnp.float32)
    asrc_ref[...] = jnp.dot(x, wsrc_ref[...], preferred_element_type=jnp.float32)


_proj = pl.pallas_call(
    _proj_body,
    grid=(20,),
    in_specs=[
        pl.BlockSpec((512, D), lambda i: (i, 0)),
        pl.BlockSpec((D, WIDTH), lambda i: (0, 0)),
        pl.BlockSpec((D, 16), lambda i: (0, 0)),
    ],
    out_specs=[
        pl.BlockSpec((512, WIDTH), lambda i: (i, 0)),
        pl.BlockSpec((512, 16), lambda i: (i, 0)),
    ],
    out_shape=[
        jax.ShapeDtypeStruct((NPAD, WIDTH), jnp.float32),
        jax.ShapeDtypeStruct((NPAD, 16), jnp.float32),
    ],
)


@functools.cache
def _make_edge_kernel():
    mesh = plsc.VectorSubcoreMesh(
        core_axis_name="c", subcore_axis_name="s",
        num_cores=NC, num_subcores=NS)
    return pl.kernel(
        _edge_body,
        out_type=jax.ShapeDtypeStruct((NC, NPAD, WIDTH), jnp.float32),
        mesh=mesh,
        scratch_types=[
            pltpu.VMEM_SHARED((NPAD, WIDTH), jnp.float32),
            pltpu.VMEM((CH, 16), jnp.float32),
            pltpu.VMEM((CH,), jnp.int32),
            pltpu.VMEM((CH,), jnp.int32),
            pltpu.VMEM((CH, WIDTH), jnp.float32),
            pltpu.SemaphoreType.DMA,
            pltpu.SemaphoreType.DMA,
        ],
        compiler_params=pltpu.CompilerParams(
            needs_layout_passes=False, use_tc_tiling_on_sc=False),
    )


def _edge_body(hext_hbm, asrc_hbm, esrc_hbm, edst_hbm, out_hbm,
               acc, AS, src_v, dst_v, S, sem, sem2):
    c = lax.axis_index("c")
    s = lax.axis_index("s")
    wid = c * NS + s

    zero16 = jnp.zeros((16,), jnp.float32)

    @pl.loop(0, CH)
    def _zrow(e):
        for kk in range(WIDTH // 16):
            S[e, pl.ds(kk * 16, 16)] = zero16

    @pl.loop(0, ROWS_PER_TILE // CH)
    def _zacc(t):
        pltpu.sync_copy(S, acc.at[pl.ds(s * ROWS_PER_TILE + t * CH, CH)])

    plsc.subcore_barrier()

    @pl.loop(0, NCHUNK)
    def _chunk(i):
        base = wid * EW + i * CH
        pltpu.sync_copy(esrc_hbm.at[pl.ds(base, CH)], src_v)
        pltpu.sync_copy(edst_hbm.at[pl.ds(base, CH)], dst_v)
        cp1 = pltpu.async_copy(hext_hbm.at[dst_v], S, sem)
        cp2 = pltpu.async_copy(asrc_hbm.at[src_v], AS, sem2)
        cp1.wait()
        cp2.wait()

        for g in range(CH // 16):
            e16 = lax.iota(jnp.int32, 16) + g * 16
            for h in range(H):
                col = jnp.full((16,), 128 + h, jnp.int32)
                a_s = plsc.load_gather(AS, [e16, jnp.full((16,), h, jnp.int32)])
                a_d = plsc.load_gather(S, [e16, col])
                logit = a_s + a_d
                w = jnp.exp(-jnp.maximum(logit, 0.2 * logit))
                plsc.store_scatter(S, [e16, col], w)

        @pl.loop(0, CH)
        def _row(e):
            wrow = S[e, pl.ds(HL, 16)]
            for h in range(H):
                wv = lax.broadcast(wrow[h], (16,))
                for half in range(2):
                    sl = pl.ds(h * L + half * 16, 16)
                    S[e, sl] = S[e, sl] * wv

        pltpu.sync_copy(S, acc.at[src_v], add=True)

    plsc.subcore_barrier()
    pltpu.sync_copy(acc.at[pl.ds(s * ROWS_PER_TILE, ROWS_PER_TILE)],
                    out_hbm.at[c, pl.ds(s * ROWS_PER_TILE, ROWS_PER_TILE)])


def _combine_body(p_ref, msg_ref):
    p = p_ref[0] + p_ref[1]
    num = p[:, :HL]
    r0 = lax.broadcasted_iota(jnp.int32, (WIDTH, HL), 0)
    r1 = lax.broadcasted_iota(jnp.int32, (WIDTH, HL), 1)
    rep = ((r0 - 128) == (r1 // L)).astype(jnp.float32)
    den = jnp.dot(p, rep, preferred_element_type=jnp.float32)
    den = jnp.where(den != 0.0, den, 1.0)
    m = num / den
    msg_ref[...] = jnp.where(m > 0.0, m, jnp.exp(m) - 1.0)


_combine = pl.pallas_call(
    _combine_body,
    grid=(20,),
    in_specs=[pl.BlockSpec((NC, 512, WIDTH), lambda i: (0, i, 0))],
    out_specs=pl.BlockSpec((512, HL), lambda i: (i, 0)),
    out_shape=jax.ShapeDtypeStruct((NPAD, HL), jnp.float32),
)


def _head_body(msg_ref, sc_ref, c1w_ref, b1_ref, w2r_ref, b2_ref,
               out_ref, pooled_ref):
    nrows = NPAD // 128
    r_iota = lax.broadcasted_iota(jnp.int32, (nrows, 128), 0)
    l_iota = lax.broadcasted_iota(jnp.int32, (nrows, 128), 1)
    nid = r_iota * 128 + l_iota
    vals0 = jnp.where(nid < N, sc_ref[...], -jnp.inf)
    pooled_ref[...] = jnp.zeros((32, 128), jnp.float32)

    def body(i, vals):
        m = jnp.max(vals)
        idx = jnp.min(jnp.where(vals == m, nid, jnp.int32(2**30)))
        pooled_ref[pl.ds(i, 1), :] = msg_ref[pl.ds(idx, 1), :]
        return jnp.where(nid == idx, -jnp.inf, vals)

    lax.fori_loop(0, K, body, vals0)

    pooled = pooled_ref[...]
    x1t = lax.dot_general(c1w_ref[...], pooled, (((1,), (1,)), ((), ())),
                          preferred_element_type=jnp.float32)
    x1t = jnp.maximum(x1t + b1_ref[...], 0.0)

    j_i = lax.broadcasted_iota(jnp.int32, (32, K // 2), 0)
    p_i = lax.broadcasted_iota(jnp.int32, (32, K // 2), 1)
    se = (j_i == 2 * p_i).astype(jnp.float32)
    so = (j_i == 2 * p_i + 1).astype(jnp.float32)
    xp = jnp.maximum(jnp.dot(x1t, se, preferred_element_type=jnp.float32),
                     jnp.dot(x1t, so, preferred_element_type=jnp.float32))

    npos = K // 2 - KW2 + 1
    y = jnp.zeros((C2, npos), jnp.float32)
    for t in range(KW2):
        w2t = w2r_ref[pl.ds(t * C2, C2), :]
        q_i = lax.broadcasted_iota(jnp.int32, (K // 2, npos), 0)
        s_i = lax.broadcasted_iota(jnp.int32, (K // 2, npos), 1)
        sel = (q_i == s_i + t).astype(jnp.float32)
        xpt = jnp.dot(xp, sel, preferred_element_type=jnp.float32)
        y = y + jnp.dot(w2t, xpt, preferred_element_type=jnp.float32)
    y = jnp.maximum(y + b2_ref[...], 0.0)
    out_ref[...] = jnp.concatenate(
        [y, jnp.zeros((C2, 128 - npos), jnp.float32)], axis=1)


_head = pl.pallas_call(
    _head_body,
    in_specs=[
        pl.BlockSpec(memory_space=pltpu.VMEM),
        pl.BlockSpec(memory_space=pltpu.VMEM),
        pl.BlockSpec(memory_space=pltpu.VMEM),
        pl.BlockSpec(memory_space=pltpu.VMEM),
        pl.BlockSpec(memory_space=pltpu.VMEM),
        pl.BlockSpec(memory_space=pltpu.VMEM),
    ],
    out_shape=jax.ShapeDtypeStruct((C2, 128), jnp.float32),
    scratch_shapes=[pltpu.VMEM((32, 128), jnp.float32)],
)


def kernel(node_feat, edge_index, W, a, conv1_w, conv1_b, conv2_w, conv2_b):
    f32 = jnp.float32
    wflat = W.transpose(1, 0, 2).reshape(D, HL)
    eye = jnp.eye(H, dtype=f32)
    a_src = a[:, :L]
    a_dst = a[:, L:]
    asrc_m = (a_src[:, :, None] * eye[:, None, :]).reshape(HL, H)
    adst_m = (a_dst[:, :, None] * eye[:, None, :]).reshape(HL, H)
    wext = jnp.concatenate(
        [wflat, wflat @ adst_m, jnp.zeros((D, WIDTH - HL - H), f32)], axis=1)
    wsrc = jnp.concatenate(
        [wflat @ asrc_m, jnp.zeros((D, 12), f32)], axis=1)

    node_pad = jnp.pad(node_feat, ((0, NPAD - N), (0, 0)))
    esrc = edge_index[0]
    edst = edge_index[1]

    hext, asrc = _proj(node_pad, wext, wsrc)
    partial = _make_edge_kernel()(hext, asrc, esrc, edst)
    msg = _combine(partial)
    sc2 = msg[:, HL - 1].reshape(NPAD // 128, 128)

    c1w = conv1_w[:, 0, :]
    b1 = conv1_b.reshape(C1, 1)
    w2r = conv2_w.transpose(2, 0, 1).reshape(KW2 * C2, C1)
    b2 = conv2_b.reshape(C2, 1)
    yfull = _head(msg, sc2, c1w, b1, w2r, b2)
    npos = K // 2 - KW2 + 1
    return yfull[:, :npos].reshape(1, C2 * npos)


# speedup vs baseline: 18.9659x; 1.2832x over previous
import functools

import jax
import jax.numpy as jnp
from jax import lax
from jax.experimental import pallas as pl
from jax.experimental.pallas import tpu as pltpu
from jax.experimental.pallas import tpu_sc as plsc

N = 10000
E = 320000
D = 128
L = 32
H = 4
HL = H * L
K = 30
C1 = 16
C2 = 32
KW2 = 5

NPAD = 10240
WIDTH = 144
NC, NS = 2, 16
NW = NC * NS
EW = E // NW
CH = 80
NCHUNK = EW // CH
ROWS_PER_TILE = NPAD // NS


def _proj_body(x_ref, wext_ref, wsrc_ref, hext_ref, asrc_ref):
    x = x_ref[...]
    hext_ref[...] = jnp.dot(x, wext_ref[...], preferred_element_type=jnp.float32)
    asrc_ref[...] = jnp.dot(x, wsrc_ref[...], preferred_element_type=jnp.float32)


_proj = pl.pallas_call(
    _proj_body,
    grid=(20,),
    in_specs=[
        pl.BlockSpec((512, D), lambda i: (i, 0)),
        pl.BlockSpec((D, WIDTH), lambda i: (0, 0)),
        pl.BlockSpec((D, 16), lambda i: (0, 0)),
    ],
    out_specs=[
        pl.BlockSpec((512, WIDTH), lambda i: (i, 0)),
        pl.BlockSpec((512, 16), lambda i: (i, 0)),
    ],
    out_shape=[
        jax.ShapeDtypeStruct((NPAD, WIDTH), jnp.float32),
        jax.ShapeDtypeStruct((NPAD, 16), jnp.float32),
    ],
)


@functools.cache
def _make_edge_kernel():
    mesh = plsc.VectorSubcoreMesh(
        core_axis_name="c", subcore_axis_name="s",
        num_cores=NC, num_subcores=NS)
    return pl.kernel(
        _edge_body,
        out_type=jax.ShapeDtypeStruct((NC, NPAD, WIDTH), jnp.float32),
        mesh=mesh,
        scratch_types=[
            pltpu.VMEM_SHARED((NPAD, WIDTH), jnp.float32),
            pltpu.VMEM((CH, 16), jnp.float32),
            pltpu.VMEM((CH, 16), jnp.float32),
            pltpu.VMEM((CH,), jnp.int32),
            pltpu.VMEM((CH,), jnp.int32),
            pltpu.VMEM((CH,), jnp.int32),
            pltpu.VMEM((CH,), jnp.int32),
            pltpu.VMEM((CH, WIDTH), jnp.float32),
            pltpu.VMEM((CH, WIDTH), jnp.float32),
            pltpu.SemaphoreType.DMA,
            pltpu.SemaphoreType.DMA,
        ],
        compiler_params=pltpu.CompilerParams(
            needs_layout_passes=False, use_tc_tiling_on_sc=False),
    )


def _edge_body(hext_hbm, asrc_hbm, eidx_hbm, out_hbm,
               acc, AS0, AS1, x0, x1, src0, src1, S0, S1, gsem0, gsem1):
    c = lax.axis_index("c")
    s = lax.axis_index("s")
    wid = c * NS + s
    chunk0 = wid * NCHUNK

    zero16 = jnp.zeros((16,), jnp.float32)

    @pl.loop(0, CH)
    def _zrow(e):
        for kk in range(WIDTH // 16):
            S0[e, pl.ds(kk * 16, 16)] = zero16

    @pl.loop(0, ROWS_PER_TILE // CH)
    def _zacc(t):
        pltpu.sync_copy(S0, acc.at[pl.ds(s * ROWS_PER_TILE + t * CH, CH)])

    plsc.subcore_barrier()

    def issue(i, S, AS, x, src_v, gsem):
        pltpu.sync_copy(eidx_hbm.at[chunk0 + i, 0], src_v)
        pltpu.sync_copy(eidx_hbm.at[chunk0 + i, 1], x)
        pltpu.async_copy(hext_hbm.at[x], S, gsem)
        pltpu.async_copy(asrc_hbm.at[src_v], AS, gsem)

    def drain(S, AS, x, src_v, gsem):
        pltpu.make_async_copy(hext_hbm.at[x], S, gsem).wait()
        pltpu.make_async_copy(asrc_hbm.at[src_v], AS, gsem).wait()

    def process(S, AS, src_v):
        for g in range(CH // 16):
            e16 = lax.iota(jnp.int32, 16) + g * 16
            for h in range(H):
                col = jnp.full((16,), 128 + h, jnp.int32)
                a_s = plsc.load_gather(AS, [e16, jnp.full((16,), h, jnp.int32)])
                a_d = plsc.load_gather(S, [e16, col])
                logit = a_s + a_d
                w = jnp.exp(-jnp.maximum(logit, 0.2 * logit))
                plsc.store_scatter(S, [e16, col], w)

        @pl.loop(0, CH)
        def _row(e):
            wrow = S[e, pl.ds(HL, 16)]
            for h in range(H):
                wv = lax.broadcast(wrow[h], (16,))
                for half in range(2):
                    sl = pl.ds(h * L + half * 16, 16)
                    S[e, sl] = S[e, sl] * wv

        pltpu.sync_copy(S, acc.at[src_v], add=True)

    issue(0, S0, AS0, x0, src0, gsem0)

    @pl.loop(0, (NCHUNK - 1) // 2)
    def _pair(p):
        i = 2 * p
        drain(S0, AS0, x0, src0, gsem0)
        issue(i + 1, S1, AS1, x1, src1, gsem1)
        process(S0, AS0, src0)
        drain(S1, AS1, x1, src1, gsem1)
        issue(i + 2, S0, AS0, x0, src0, gsem0)
        process(S1, AS1, src1)

    drain(S0, AS0, x0, src0, gsem0)
    process(S0, AS0, src0)

    plsc.subcore_barrier()
    pltpu.sync_copy(acc.at[pl.ds(s * ROWS_PER_TILE, ROWS_PER_TILE)],
                    out_hbm.at[c, pl.ds(s * ROWS_PER_TILE, ROWS_PER_TILE)])


def _combine_body(p_ref, msg_ref):
    p = p_ref[0] + p_ref[1]
    num = p[:, :HL]
    r0 = lax.broadcasted_iota(jnp.int32, (WIDTH, HL), 0)
    r1 = lax.broadcasted_iota(jnp.int32, (WIDTH, HL), 1)
    rep = ((r0 - 128) == (r1 // L)).astype(jnp.float32)
    den = jnp.dot(p, rep, preferred_element_type=jnp.float32)
    den = jnp.where(den != 0.0, den, 1.0)
    m = num / den
    msg_ref[...] = jnp.where(m > 0.0, m, jnp.exp(m) - 1.0)


_combine = pl.pallas_call(
    _combine_body,
    grid=(20,),
    in_specs=[pl.BlockSpec((NC, 512, WIDTH), lambda i: (0, i, 0))],
    out_specs=pl.BlockSpec((512, HL), lambda i: (i, 0)),
    out_shape=jax.ShapeDtypeStruct((NPAD, HL), jnp.float32),
)


def _head_body(msg_ref, sc_ref, c1w_ref, b1_ref, w2r_ref, b2_ref,
               out_ref, pooled_ref):
    nrows = NPAD // 128
    r_iota = lax.broadcasted_iota(jnp.int32, (nrows, 128), 0)
    l_iota = lax.broadcasted_iota(jnp.int32, (nrows, 128), 1)
    nid = r_iota * 128 + l_iota
    vals0 = jnp.where(nid < N, sc_ref[...], -jnp.inf)
    pooled_ref[...] = jnp.zeros((32, 128), jnp.float32)

    def body(i, vals):
        m = jnp.max(vals)
        idx = jnp.min(jnp.where(vals == m, nid, jnp.int32(2**30)))
        pooled_ref[pl.ds(i, 1), :] = msg_ref[pl.ds(idx, 1), :]
        return jnp.where(nid == idx, -jnp.inf, vals)

    lax.fori_loop(0, K, body, vals0)

    pooled = pooled_ref[...]
    x1t = lax.dot_general(c1w_ref[...], pooled, (((1,), (1,)), ((), ())),
                          preferred_element_type=jnp.float32)
    x1t = jnp.maximum(x1t + b1_ref[...], 0.0)

    j_i = lax.broadcasted_iota(jnp.int32, (32, K // 2), 0)
    p_i = lax.broadcasted_iota(jnp.int32, (32, K // 2), 1)
    se = (j_i == 2 * p_i).astype(jnp.float32)
    so = (j_i == 2 * p_i + 1).astype(jnp.float32)
    xp = jnp.maximum(jnp.dot(x1t, se, preferred_element_type=jnp.float32),
                     jnp.dot(x1t, so, preferred_element_type=jnp.float32))

    npos = K // 2 - KW2 + 1
    y = jnp.zeros((C2, npos), jnp.float32)
    for t in range(KW2):
        w2t = w2r_ref[pl.ds(t * C2, C2), :]
        q_i = lax.broadcasted_iota(jnp.int32, (K // 2, npos), 0)
        s_i = lax.broadcasted_iota(jnp.int32, (K // 2, npos), 1)
        sel = (q_i == s_i + t).astype(jnp.float32)
        xpt = jnp.dot(xp, sel, preferred_element_type=jnp.float32)
        y = y + jnp.dot(w2t, xpt, preferred_element_type=jnp.float32)
    y = jnp.maximum(y + b2_ref[...], 0.0)
    out_ref[...] = jnp.concatenate(
        [y, jnp.zeros((C2, 128 - npos), jnp.float32)], axis=1)


_head = pl.pallas_call(
    _head_body,
    in_specs=[
        pl.BlockSpec(memory_space=pltpu.VMEM),
        pl.BlockSpec(memory_space=pltpu.VMEM),
        pl.BlockSpec(memory_space=pltpu.VMEM),
        pl.BlockSpec(memory_space=pltpu.VMEM),
        pl.BlockSpec(memory_space=pltpu.VMEM),
        pl.BlockSpec(memory_space=pltpu.VMEM),
    ],
    out_shape=jax.ShapeDtypeStruct((C2, 128), jnp.float32),
    scratch_shapes=[pltpu.VMEM((32, 128), jnp.float32)],
)


def kernel(node_feat, edge_index, W, a, conv1_w, conv1_b, conv2_w, conv2_b):
    f32 = jnp.float32
    wflat = W.transpose(1, 0, 2).reshape(D, HL)
    eye = jnp.eye(H, dtype=f32)
    a_src = a[:, :L]
    a_dst = a[:, L:]
    asrc_m = (a_src[:, :, None] * eye[:, None, :]).reshape(HL, H)
    adst_m = (a_dst[:, :, None] * eye[:, None, :]).reshape(HL, H)
    wext = jnp.concatenate(
        [wflat, wflat @ adst_m, jnp.zeros((D, WIDTH - HL - H), f32)], axis=1)
    wsrc = jnp.concatenate(
        [wflat @ asrc_m, jnp.zeros((D, 12), f32)], axis=1)

    node_pad = jnp.pad(node_feat, ((0, NPAD - N), (0, 0)))
    epacked = edge_index.reshape(2, E // CH, CH).transpose(1, 0, 2)

    hext, asrc = _proj(node_pad, wext, wsrc)
    partial = _make_edge_kernel()(hext, asrc, epacked)
    msg = _combine(partial)
    sc2 = msg[:, HL - 1].reshape(NPAD // 128, 128)

    c1w = conv1_w[:, 0, :]
    b1 = conv1_b.reshape(C1, 1)
    w2r = conv2_w.transpose(2, 0, 1).reshape(KW2 * C2, C1)
    b2 = conv2_b.reshape(C2, 1)
    yfull = _head(msg, sc2, c1w, b1, w2r, b2)
    npos = K // 2 - KW2 + 1
    return yfull[:, :npos].reshape(1, C2 * npos)


# speedup vs baseline: 19.3911x; 1.0224x over previous
import functools

import jax
import jax.numpy as jnp
from jax import lax
from jax.experimental import pallas as pl
from jax.experimental.pallas import tpu as pltpu
from jax.experimental.pallas import tpu_sc as plsc

N = 10000
E = 320000
D = 128
L = 32
H = 4
HL = H * L
K = 30
C1 = 16
C2 = 32
KW2 = 5

NPAD = 10240
WIDTH = 144
NC, NS = 2, 16
NW = NC * NS
EW = E // NW
CH = 80
NCHUNK = EW // CH
ROWS_PER_TILE = NPAD // NS


def _proj_body(x_ref, wflat_ref, adst_ref, asrcm_ref, hext_ref, asrc_ref):
    x = x_ref[...]
    h = jnp.dot(x, wflat_ref[...], preferred_element_type=jnp.float32)
    ad = jnp.dot(h, adst_ref[...], preferred_element_type=jnp.float32)
    hext_ref[...] = jnp.concatenate([h, ad], axis=1)
    asrc_ref[...] = jnp.dot(h, asrcm_ref[...], preferred_element_type=jnp.float32)


_proj = pl.pallas_call(
    _proj_body,
    grid=(20,),
    in_specs=[
        pl.BlockSpec((512, D), lambda i: (i, 0)),
        pl.BlockSpec((D, D), lambda i: (0, 0)),
        pl.BlockSpec((HL, 16), lambda i: (0, 0)),
        pl.BlockSpec((HL, 16), lambda i: (0, 0)),
    ],
    out_specs=[
        pl.BlockSpec((512, WIDTH), lambda i: (i, 0)),
        pl.BlockSpec((512, 16), lambda i: (i, 0)),
    ],
    out_shape=[
        jax.ShapeDtypeStruct((NPAD, WIDTH), jnp.float32),
        jax.ShapeDtypeStruct((NPAD, 16), jnp.float32),
    ],
)


@functools.cache
def _make_edge_kernel():
    mesh = plsc.VectorSubcoreMesh(
        core_axis_name="c", subcore_axis_name="s",
        num_cores=NC, num_subcores=NS)
    return pl.kernel(
        _edge_body,
        out_type=jax.ShapeDtypeStruct((NC, NPAD, WIDTH), jnp.float32),
        mesh=mesh,
        scratch_types=[
            pltpu.VMEM_SHARED((NPAD, WIDTH), jnp.float32),
            [pltpu.VMEM((CH, WIDTH), jnp.float32) for _ in range(3)],
            [pltpu.VMEM((CH, 16), jnp.float32) for _ in range(3)],
            [pltpu.VMEM((CH,), jnp.int32) for _ in range(3)],
            [pltpu.VMEM((CH,), jnp.int32) for _ in range(3)],
            [pltpu.SemaphoreType.DMA for _ in range(3)],
            [pltpu.SemaphoreType.DMA for _ in range(3)],
        ],
        compiler_params=pltpu.CompilerParams(
            needs_layout_passes=False, use_tc_tiling_on_sc=False),
    )


def _edge_body(hext_hbm, asrc_hbm, esrc_hbm, edst_hbm, out_hbm,
               acc, Ss, ASs, srcs, dsts, gsems, ssems):
    c = lax.axis_index("c")
    s = lax.axis_index("s")
    wid = c * NS + s
    ebase = wid * EW

    zero16 = jnp.zeros((16,), jnp.float32)
    S0 = Ss[0]

    @pl.loop(0, CH)
    def _zrow(e):
        for kk in range(WIDTH // 16):
            S0[e, pl.ds(kk * 16, 16)] = zero16

    @pl.loop(0, ROWS_PER_TILE // CH)
    def _zacc(t):
        pltpu.sync_copy(S0, acc.at[pl.ds(s * ROWS_PER_TILE + t * CH, CH)])

    plsc.subcore_barrier()

    def issue(i, b):
        base = ebase + i * CH
        pltpu.sync_copy(esrc_hbm.at[pl.ds(base, CH)], srcs[b])
        pltpu.sync_copy(edst_hbm.at[pl.ds(base, CH)], dsts[b])
        pltpu.async_copy(hext_hbm.at[dsts[b]], Ss[b], gsems[b])
        pltpu.async_copy(asrc_hbm.at[srcs[b]], ASs[b], gsems[b])

    def drain_gather(b):
        pltpu.make_async_copy(hext_hbm.at[dsts[b]], Ss[b], gsems[b]).wait()
        pltpu.make_async_copy(asrc_hbm.at[srcs[b]], ASs[b], gsems[b]).wait()

    def wait_scatter(b):
        pass

    def compute(b):
        S, AS = Ss[b], ASs[b]
        for g in range(CH // 16):
            e16 = lax.iota(jnp.int32, 16) + g * 16
            for h in range(H):
                col = jnp.full((16,), 128 + h, jnp.int32)
                a_s = plsc.load_gather(AS, [e16, jnp.full((16,), h, jnp.int32)])
                a_d = plsc.load_gather(S, [e16, col])
                logit = a_s + a_d
                w = jnp.exp(-jnp.maximum(logit, 0.2 * logit))
                plsc.store_scatter(S, [e16, col], w)

        @pl.loop(0, CH)
        def _row(e):
            wrow = S[e, pl.ds(HL, 16)]
            for h in range(H):
                wv = lax.broadcast(wrow[h], (16,))
                for half in range(2):
                    sl = pl.ds(h * L + half * 16, 16)
                    S[e, sl] = S[e, sl] * wv

    def scatter(b):
        pltpu.sync_copy(Ss[b], acc.at[srcs[b]], add=True)

    issue(0, 0)
    issue(1, 1)
    drain_gather(0)
    compute(0)
    issue(2, 2)
    scatter(0)
    drain_gather(1)
    compute(1)
    wait_scatter(0)
    issue(3, 0)
    scatter(1)

    @pl.loop(0, (NCHUNK - 5) // 3)
    def _triple(p):
        j = 3 * p + 2
        for k in range(3):
            b = (2 + k) % 3
            b2 = (4 + k) % 3
            drain_gather(b)
            compute(b)
            wait_scatter(b2)
            issue(j + k + 2, b2)
            scatter(b)

    _j = NCHUNK - 3
    drain_gather(_j % 3)
    compute(_j % 3)
    wait_scatter((_j + 2) % 3)
    issue(_j + 2, (_j + 2) % 3)
    scatter(_j % 3)
    for _j in range(NCHUNK - 2, NCHUNK):
        drain_gather(_j % 3)
        compute(_j % 3)
        scatter(_j % 3)
    for _b in range(3):
        wait_scatter(_b)

    plsc.subcore_barrier()
    pltpu.sync_copy(acc.at[pl.ds(s * ROWS_PER_TILE, ROWS_PER_TILE)],
                    out_hbm.at[c, pl.ds(s * ROWS_PER_TILE, ROWS_PER_TILE)])


def _combine_body(p_ref, msg_ref, scv_ref):
    p = p_ref[0] + p_ref[1]
    num = p[:, :HL]
    r0 = lax.broadcasted_iota(jnp.int32, (WIDTH, HL), 0)
    r1 = lax.broadcasted_iota(jnp.int32, (WIDTH, HL), 1)
    rep = ((r0 - 128) == (r1 // L)).astype(jnp.float32)
    den = jnp.dot(p, rep, preferred_element_type=jnp.float32,
                  precision=lax.Precision.HIGHEST)
    den = jnp.where(den != 0.0, den, 1.0)
    m = num / den
    m = jnp.where(m > 0.0, m, jnp.exp(m) - 1.0)
    msg_ref[...] = m
    scv_ref[...] = m[:, HL - 1:HL]


_combine = pl.pallas_call(
    _combine_body,
    grid=(20,),
    in_specs=[pl.BlockSpec((NC, 512, WIDTH), lambda i: (0, i, 0))],
    out_specs=[
        pl.BlockSpec((512, HL), lambda i: (i, 0)),
        pl.BlockSpec((512, 1), lambda i: (i, 0)),
    ],
    out_shape=[
        jax.ShapeDtypeStruct((NPAD, HL), jnp.float32),
        jax.ShapeDtypeStruct((NPAD, 1), jnp.float32),
    ],
)


def _head_body(msg_ref, sc_ref, c1w_ref, b1_ref, w2r_ref, b2_ref,
               out_ref, pooled_ref):
    nrows = NPAD // 128
    r_iota = lax.broadcasted_iota(jnp.int32, (nrows, 128), 0)
    l_iota = lax.broadcasted_iota(jnp.int32, (nrows, 128), 1)
    nid = r_iota * 128 + l_iota
    vals0 = jnp.where(nid < N, sc_ref[...], -jnp.inf)
    pooled_ref[...] = jnp.zeros((32, 128), jnp.float32)

    def body(i, vals):
        m = jnp.max(vals)
        idx = jnp.min(jnp.where(vals == m, nid, jnp.int32(2**30)))
        pooled_ref[pl.ds(i, 1), :] = msg_ref[pl.ds(idx, 1), :]
        return jnp.where(nid == idx, -jnp.inf, vals)

    lax.fori_loop(0, K, body, vals0)

    pooled = pooled_ref[...]
    x1t = lax.dot_general(c1w_ref[...], pooled, (((1,), (1,)), ((), ())),
                          preferred_element_type=jnp.float32)
    x1t = jnp.maximum(x1t + b1_ref[...], 0.0)

    j_i = lax.broadcasted_iota(jnp.int32, (32, K // 2), 0)
    p_i = lax.broadcasted_iota(jnp.int32, (32, K // 2), 1)
    se = (j_i == 2 * p_i).astype(jnp.float32)
    so = (j_i == 2 * p_i + 1).astype(jnp.float32)
    xp = jnp.maximum(
        jnp.dot(x1t, se, preferred_element_type=jnp.float32,
                precision=lax.Precision.HIGHEST),
        jnp.dot(x1t, so, preferred_element_type=jnp.float32,
                precision=lax.Precision.HIGHEST))

    npos = K // 2 - KW2 + 1
    y = jnp.zeros((C2, npos), jnp.float32)
    for t in range(KW2):
        w2t = w2r_ref[pl.ds(t * C2, C2), :]
        q_i = lax.broadcasted_iota(jnp.int32, (K // 2, npos), 0)
        s_i = lax.broadcasted_iota(jnp.int32, (K // 2, npos), 1)
        sel = (q_i == s_i + t).astype(jnp.float32)
        xpt = jnp.dot(xp, sel, preferred_element_type=jnp.float32,
                      precision=lax.Precision.HIGHEST)
        y = y + jnp.dot(w2t, xpt, preferred_element_type=jnp.float32)
    y = jnp.maximum(y + b2_ref[...], 0.0)
    out_ref[...] = jnp.concatenate(
        [y, jnp.zeros((C2, 128 - npos), jnp.float32)], axis=1)


_head = pl.pallas_call(
    _head_body,
    in_specs=[
        pl.BlockSpec(memory_space=pltpu.VMEM),
        pl.BlockSpec(memory_space=pltpu.VMEM),
        pl.BlockSpec(memory_space=pltpu.VMEM),
        pl.BlockSpec(memory_space=pltpu.VMEM),
        pl.BlockSpec(memory_space=pltpu.VMEM),
        pl.BlockSpec(memory_space=pltpu.VMEM),
    ],
    out_shape=jax.ShapeDtypeStruct((C2, 128), jnp.float32),
    scratch_shapes=[pltpu.VMEM((32, 128), jnp.float32)],
)


def kernel(node_feat, edge_index, W, a, conv1_w, conv1_b, conv2_w, conv2_b):
    f32 = jnp.float32
    wflat = W.transpose(1, 0, 2).reshape(D, HL)
    eye = jnp.eye(H, dtype=f32)
    a_src = a[:, :L]
    a_dst = a[:, L:]
    asrc_m = (a_src[:, :, None] * eye[:, None, :]).reshape(HL, H)
    adst_m = (a_dst[:, :, None] * eye[:, None, :]).reshape(HL, H)
    adst16 = jnp.concatenate([adst_m, jnp.zeros((HL, 12), f32)], axis=1)
    asrc16 = jnp.concatenate([asrc_m, jnp.zeros((HL, 12), f32)], axis=1)

    node_pad = jnp.pad(node_feat, ((0, NPAD - N), (0, 0)))

    hext, asrc = _proj(node_pad, wflat, adst16, asrc16)
    partial = _make_edge_kernel()(hext, asrc, edge_index[0], edge_index[1])
    msg, scv = _combine(partial)
    sc2 = scv.reshape(NPAD // 128, 128)

    c1w = conv1_w[:, 0, :]
    b1 = conv1_b.reshape(C1, 1)
    w2r = conv2_w.transpose(2, 0, 1).reshape(KW2 * C2, C1)
    b2 = conv2_b.reshape(C2, 1)
    yfull = _head(msg, sc2, c1w, b1, w2r, b2)
    npos = K // 2 - KW2 + 1
    return yfull[:, :npos].reshape(1, C2 * npos)


# speedup vs baseline: 22.3199x; 1.1510x over previous
import functools

import jax
import jax.numpy as jnp
from jax import lax
from jax.experimental import pallas as pl
from jax.experimental.pallas import tpu as pltpu
from jax.experimental.pallas import tpu_sc as plsc

N = 10000
E = 320000
D = 128
L = 32
H = 4
HL = H * L
K = 30
C1 = 16
C2 = 32
KW2 = 5

NPAD = 10240
WIDTH = 144
NC, NS = 2, 16
NW = NC * NS
EW = E // NW
CH = 80
NCHUNK = EW // CH
ROWS_PER_TILE = NPAD // NS


def _proj_body(x_ref, wflat_ref, adst_ref, asrcm_ref, hext_ref, asrc_ref):
    x = x_ref[...]
    h = jnp.dot(x, wflat_ref[...], preferred_element_type=jnp.float32)
    ad = jnp.dot(h, adst_ref[...], preferred_element_type=jnp.float32)
    hext_ref[...] = jnp.concatenate([h, ad], axis=1)
    asrc_ref[...] = jnp.dot(h, asrcm_ref[...], preferred_element_type=jnp.float32)


_proj = pl.pallas_call(
    _proj_body,
    grid=(20,),
    in_specs=[
        pl.BlockSpec((512, D), lambda i: (i, 0)),
        pl.BlockSpec((D, D), lambda i: (0, 0)),
        pl.BlockSpec((HL, 16), lambda i: (0, 0)),
        pl.BlockSpec((HL, 16), lambda i: (0, 0)),
    ],
    out_specs=[
        pl.BlockSpec((512, WIDTH), lambda i: (i, 0)),
        pl.BlockSpec((512, 16), lambda i: (i, 0)),
    ],
    out_shape=[
        jax.ShapeDtypeStruct((NPAD, WIDTH), jnp.float32),
        jax.ShapeDtypeStruct((NPAD, 16), jnp.float32),
    ],
)


@functools.cache
def _make_edge_kernel():
    mesh = plsc.VectorSubcoreMesh(
        core_axis_name="c", subcore_axis_name="s",
        num_cores=NC, num_subcores=NS)
    return pl.kernel(
        _edge_body,
        out_type=jax.ShapeDtypeStruct((NC, NPAD, WIDTH), jnp.float32),
        mesh=mesh,
        scratch_types=[
            pltpu.VMEM_SHARED((NPAD, WIDTH), jnp.float32),
            [pltpu.VMEM((CH, WIDTH), jnp.float32) for _ in range(3)],
            [pltpu.VMEM((CH, 16), jnp.float32) for _ in range(3)],
            [pltpu.VMEM((CH,), jnp.int32) for _ in range(3)],
            [pltpu.VMEM((CH,), jnp.int32) for _ in range(3)],
            [pltpu.SemaphoreType.DMA for _ in range(3)],
            [pltpu.SemaphoreType.DMA for _ in range(3)],
        ],
        compiler_params=pltpu.CompilerParams(
            needs_layout_passes=False, use_tc_tiling_on_sc=False),
    )


def _edge_body(hext_hbm, asrc_hbm, esrc_hbm, edst_hbm, out_hbm,
               acc, Ss, ASs, srcs, dsts, gsems, ssems):
    c = lax.axis_index("c")
    s = lax.axis_index("s")
    wid = c * NS + s
    ebase = wid * EW

    zero16 = jnp.zeros((16,), jnp.float32)
    S0 = Ss[0]

    @pl.loop(0, CH)
    def _zrow(e):
        for kk in range(WIDTH // 16):
            S0[e, pl.ds(kk * 16, 16)] = zero16

    @pl.loop(0, ROWS_PER_TILE // CH)
    def _zacc(t):
        pltpu.sync_copy(S0, acc.at[pl.ds(s * ROWS_PER_TILE + t * CH, CH)])

    plsc.subcore_barrier()

    def issue(i, b):
        base = ebase + i * CH
        pltpu.sync_copy(esrc_hbm.at[pl.ds(base, CH)], srcs[b])
        pltpu.sync_copy(edst_hbm.at[pl.ds(base, CH)], dsts[b])
        pltpu.async_copy(hext_hbm.at[dsts[b]], Ss[b], gsems[b])
        pltpu.async_copy(asrc_hbm.at[srcs[b]], ASs[b], gsems[b])

    def drain_gather(b):
        pltpu.make_async_copy(hext_hbm.at[dsts[b]], Ss[b], gsems[b]).wait()
        pltpu.make_async_copy(asrc_hbm.at[srcs[b]], ASs[b], gsems[b]).wait()

    def wait_scatter(b):
        pltpu.make_async_copy(Ss[b], acc.at[srcs[b]], ssems[b]).wait()

    def compute(b):
        S, AS = Ss[b], ASs[b]
        for g in range(CH // 16):
            e16 = lax.iota(jnp.int32, 16) + g * 16
            for h in range(H):
                col = jnp.full((16,), 128 + h, jnp.int32)
                a_s = plsc.load_gather(AS, [e16, jnp.full((16,), h, jnp.int32)])
                a_d = plsc.load_gather(S, [e16, col])
                logit = a_s + a_d
                w = jnp.exp(-jnp.maximum(logit, 0.2 * logit))
                plsc.store_scatter(S, [e16, col], w)

        @pl.loop(0, CH, unroll=4)
        def _row(e):
            wrow = S[e, pl.ds(HL, 16)]
            for h in range(H):
                wv = lax.broadcast(wrow[h], (16,))
                for half in range(2):
                    sl = pl.ds(h * L + half * 16, 16)
                    S[e, sl] = S[e, sl] * wv

    def scatter(b):
        pltpu.async_copy(Ss[b], acc.at[srcs[b]], ssems[b], add=True)

    issue(0, 0)
    issue(1, 1)
    drain_gather(0)
    compute(0)
    issue(2, 2)
    scatter(0)
    drain_gather(1)
    compute(1)
    wait_scatter(0)
    issue(3, 0)
    scatter(1)

    @pl.loop(0, (NCHUNK - 5) // 3)
    def _triple(p):
        j = 3 * p + 2
        for k in range(3):
            b = (2 + k) % 3
            b2 = (4 + k) % 3
            drain_gather(b)
            compute(b)
            wait_scatter(b2)
            issue(j + k + 2, b2)
            scatter(b)

    _j = NCHUNK - 3
    drain_gather(_j % 3)
    compute(_j % 3)
    wait_scatter((_j + 2) % 3)
    issue(_j + 2, (_j + 2) % 3)
    scatter(_j % 3)
    for _j in range(NCHUNK - 2, NCHUNK):
        drain_gather(_j % 3)
        compute(_j % 3)
        scatter(_j % 3)
    for _b in range(3):
        wait_scatter(_b)

    plsc.subcore_barrier()
    pltpu.sync_copy(acc.at[pl.ds(s * ROWS_PER_TILE, ROWS_PER_TILE)],
                    out_hbm.at[c, pl.ds(s * ROWS_PER_TILE, ROWS_PER_TILE)])


def _combine_body(p_ref, msg_ref, scv_ref):
    p = p_ref[0] + p_ref[1]
    num = p[:, :HL]
    r0 = lax.broadcasted_iota(jnp.int32, (WIDTH, HL), 0)
    r1 = lax.broadcasted_iota(jnp.int32, (WIDTH, HL), 1)
    rep = ((r0 - 128) == (r1 // L)).astype(jnp.float32)
    den = jnp.dot(p, rep, preferred_element_type=jnp.float32,
                  precision=lax.Precision.HIGHEST)
    den = jnp.where(den != 0.0, den, 1.0)
    m = num / den
    m = jnp.where(m > 0.0, m, jnp.exp(m) - 1.0)
    msg_ref[...] = m
    scv_ref[...] = m[:, HL - 1:HL]


_combine = pl.pallas_call(
    _combine_body,
    grid=(20,),
    in_specs=[pl.BlockSpec((NC, 512, WIDTH), lambda i: (0, i, 0))],
    out_specs=[
        pl.BlockSpec((512, HL), lambda i: (i, 0)),
        pl.BlockSpec((512, 1), lambda i: (i, 0)),
    ],
    out_shape=[
        jax.ShapeDtypeStruct((NPAD, HL), jnp.float32),
        jax.ShapeDtypeStruct((NPAD, 1), jnp.float32),
    ],
)


def _head_body(msg_ref, sc_ref, c1w_ref, b1_ref, w2r_ref, b2_ref,
               out_ref, pooled_ref):
    nrows = NPAD // 128
    r_iota = lax.broadcasted_iota(jnp.int32, (nrows, 128), 0)
    l_iota = lax.broadcasted_iota(jnp.int32, (nrows, 128), 1)
    nid = r_iota * 128 + l_iota
    vals0 = jnp.where(nid < N, sc_ref[...], -jnp.inf)
    pooled_ref[...] = jnp.zeros((32, 128), jnp.float32)

    def body(i, vals):
        m = jnp.max(vals)
        idx = jnp.min(jnp.where(vals == m, nid, jnp.int32(2**30)))
        pooled_ref[pl.ds(i, 1), :] = msg_ref[pl.ds(idx, 1), :]
        return jnp.where(nid == idx, -jnp.inf, vals)

    lax.fori_loop(0, K, body, vals0)

    pooled = pooled_ref[...]
    x1t = lax.dot_general(c1w_ref[...], pooled, (((1,), (1,)), ((), ())),
                          preferred_element_type=jnp.float32)
    x1t = jnp.maximum(x1t + b1_ref[...], 0.0)

    j_i = lax.broadcasted_iota(jnp.int32, (32, K // 2), 0)
    p_i = lax.broadcasted_iota(jnp.int32, (32, K // 2), 1)
    se = (j_i == 2 * p_i).astype(jnp.float32)
    so = (j_i == 2 * p_i + 1).astype(jnp.float32)
    xp = jnp.maximum(
        jnp.dot(x1t, se, preferred_element_type=jnp.float32,
                precision=lax.Precision.HIGHEST),
        jnp.dot(x1t, so, preferred_element_type=jnp.float32,
                precision=lax.Precision.HIGHEST))

    npos = K // 2 - KW2 + 1
    y = jnp.zeros((C2, npos), jnp.float32)
    for t in range(KW2):
        w2t = w2r_ref[pl.ds(t * C2, C2), :]
        q_i = lax.broadcasted_iota(jnp.int32, (K // 2, npos), 0)
        s_i = lax.broadcasted_iota(jnp.int32, (K // 2, npos), 1)
        sel = (q_i == s_i + t).astype(jnp.float32)
        xpt = jnp.dot(xp, sel, preferred_element_type=jnp.float32,
                      precision=lax.Precision.HIGHEST)
        y = y + jnp.dot(w2t, xpt, preferred_element_type=jnp.float32)
    y = jnp.maximum(y + b2_ref[...], 0.0)
    out_ref[...] = jnp.concatenate(
        [y, jnp.zeros((C2, 128 - npos), jnp.float32)], axis=1)


_head = pl.pallas_call(
    _head_body,
    in_specs=[
        pl.BlockSpec(memory_space=pltpu.VMEM),
        pl.BlockSpec(memory_space=pltpu.VMEM),
        pl.BlockSpec(memory_space=pltpu.VMEM),
        pl.BlockSpec(memory_space=pltpu.VMEM),
        pl.BlockSpec(memory_space=pltpu.VMEM),
        pl.BlockSpec(memory_space=pltpu.VMEM),
    ],
    out_shape=jax.ShapeDtypeStruct((C2, 128), jnp.float32),
    scratch_shapes=[pltpu.VMEM((32, 128), jnp.float32)],
)


def kernel(node_feat, edge_index, W, a, conv1_w, conv1_b, conv2_w, conv2_b):
    f32 = jnp.float32
    wflat = W.transpose(1, 0, 2).reshape(D, HL)
    eye = jnp.eye(H, dtype=f32)
    a_src = a[:, :L]
    a_dst = a[:, L:]
    asrc_m = (a_src[:, :, None] * eye[:, None, :]).reshape(HL, H)
    adst_m = (a_dst[:, :, None] * eye[:, None, :]).reshape(HL, H)
    adst16 = jnp.concatenate([adst_m, jnp.zeros((HL, 12), f32)], axis=1)
    asrc16 = jnp.concatenate([asrc_m, jnp.zeros((HL, 12), f32)], axis=1)

    node_pad = jnp.pad(node_feat, ((0, NPAD - N), (0, 0)))

    hext, asrc = _proj(node_pad, wflat, adst16, asrc16)
    partial = _make_edge_kernel()(hext, asrc, edge_index[0], edge_index[1])
    msg, scv = _combine(partial)
    sc2 = scv.reshape(NPAD // 128, 128)

    c1w = conv1_w[:, 0, :]
    b1 = conv1_b.reshape(C1, 1)
    w2r = conv2_w.transpose(2, 0, 1).reshape(KW2 * C2, C1)
    b2 = conv2_b.reshape(C2, 1)
    yfull = _head(msg, sc2, c1w, b1, w2r, b2)
    npos = K // 2 - KW2 + 1
    return yfull[:, :npos].reshape(1, C2 * npos)


# speedup vs baseline: 23.0199x; 1.0314x over previous
import functools

import jax
import jax.numpy as jnp
from jax import lax
from jax.experimental import pallas as pl
from jax.experimental.pallas import tpu as pltpu
from jax.experimental.pallas import tpu_sc as plsc

N = 10000
E = 320000
D = 128
L = 32
H = 4
HL = H * L
K = 30
C1 = 16
C2 = 32
KW2 = 5

NPAD = 10240
WIDTH = 144
NC, NS = 2, 16
NW = NC * NS
EW = E // NW
CH = 80
NCHUNK = EW // CH
ROWS_PER_TILE = NPAD // NS


def _proj_body(x_ref, wflat_ref, adst_ref, asrcm_ref, hext_ref, asrc_ref):
    x = x_ref[...]
    h = jnp.dot(x, wflat_ref[...], preferred_element_type=jnp.float32)
    ad = jnp.dot(h, adst_ref[...], preferred_element_type=jnp.float32)
    hext_ref[...] = jnp.concatenate([h, ad], axis=1)
    asrc_ref[...] = jnp.dot(h, asrcm_ref[...], preferred_element_type=jnp.float32)


_proj = pl.pallas_call(
    _proj_body,
    grid=(20,),
    in_specs=[
        pl.BlockSpec((512, D), lambda i: (i, 0)),
        pl.BlockSpec((D, D), lambda i: (0, 0)),
        pl.BlockSpec((HL, 16), lambda i: (0, 0)),
        pl.BlockSpec((HL, 16), lambda i: (0, 0)),
    ],
    out_specs=[
        pl.BlockSpec((512, WIDTH), lambda i: (i, 0)),
        pl.BlockSpec((512, 16), lambda i: (i, 0)),
    ],
    out_shape=[
        jax.ShapeDtypeStruct((NPAD, WIDTH), jnp.float32),
        jax.ShapeDtypeStruct((NPAD, 16), jnp.float32),
    ],
)


@functools.cache
def _make_edge_kernel():
    mesh = plsc.VectorSubcoreMesh(
        core_axis_name="c", subcore_axis_name="s",
        num_cores=NC, num_subcores=NS)
    return pl.kernel(
        _edge_body,
        out_type=jax.ShapeDtypeStruct((NC, NPAD, WIDTH), jnp.float32),
        mesh=mesh,
        scratch_types=[
            pltpu.VMEM_SHARED((NPAD, WIDTH), jnp.float32),
            [pltpu.VMEM((CH, WIDTH), jnp.float32) for _ in range(3)],
            [pltpu.VMEM((CH, 16), jnp.float32) for _ in range(3)],
            [pltpu.VMEM((CH,), jnp.int32) for _ in range(3)],
            [pltpu.VMEM((CH,), jnp.int32) for _ in range(3)],
            [pltpu.SemaphoreType.DMA for _ in range(3)],
            [pltpu.SemaphoreType.DMA for _ in range(3)],
        ],
        compiler_params=pltpu.CompilerParams(
            needs_layout_passes=False, use_tc_tiling_on_sc=False),
    )


def _edge_body(hext_hbm, asrc_hbm, eidx_hbm, out_hbm,
               acc, Ss, ASs, srcs, dsts, gsems, ssems):
    c = lax.axis_index("c")
    s = lax.axis_index("s")
    wid = c * NS + s
    ebase = wid * EW

    zero16 = jnp.zeros((16,), jnp.float32)
    S0 = Ss[0]

    @pl.loop(0, CH)
    def _zrow(e):
        for kk in range(WIDTH // 16):
            S0[e, pl.ds(kk * 16, 16)] = zero16

    @pl.loop(0, ROWS_PER_TILE // CH)
    def _zacc(t):
        pltpu.sync_copy(S0, acc.at[pl.ds(s * ROWS_PER_TILE + t * CH, CH)])

    plsc.subcore_barrier()

    def issue(i, b):
        base = ebase + i * CH
        pltpu.sync_copy(eidx_hbm.at[0, pl.ds(base, CH)], srcs[b])
        pltpu.sync_copy(eidx_hbm.at[1, pl.ds(base, CH)], dsts[b])
        pltpu.async_copy(hext_hbm.at[dsts[b]], Ss[b], gsems[b])
        pltpu.async_copy(asrc_hbm.at[srcs[b]], ASs[b], gsems[b])

    def drain_gather(b):
        pltpu.make_async_copy(hext_hbm.at[dsts[b]], Ss[b], gsems[b]).wait()
        pltpu.make_async_copy(asrc_hbm.at[srcs[b]], ASs[b], gsems[b]).wait()

    def wait_scatter(b):
        pltpu.make_async_copy(Ss[b], acc.at[srcs[b]], ssems[b]).wait()

    def compute(b):
        S, AS = Ss[b], ASs[b]
        for g in range(CH // 16):
            e16 = lax.iota(jnp.int32, 16) + g * 16
            for h in range(H):
                col = jnp.full((16,), 128 + h, jnp.int32)
                a_s = plsc.load_gather(AS, [e16, jnp.full((16,), h, jnp.int32)])
                a_d = plsc.load_gather(S, [e16, col])
                logit = a_s + a_d
                w = jnp.exp(-jnp.maximum(logit, 0.2 * logit))
                plsc.store_scatter(S, [e16, col], w)

        @pl.loop(0, CH, unroll=4)
        def _row(e):
            wrow = S[e, pl.ds(HL, 16)]
            for h in range(H):
                wv = lax.broadcast(wrow[h], (16,))
                for half in range(2):
                    sl = pl.ds(h * L + half * 16, 16)
                    S[e, sl] = S[e, sl] * wv

    def scatter(b):
        pltpu.async_copy(Ss[b], acc.at[srcs[b]], ssems[b], add=True)

    issue(0, 0)
    issue(1, 1)
    drain_gather(0)
    compute(0)
    issue(2, 2)
    scatter(0)
    drain_gather(1)
    compute(1)
    wait_scatter(0)
    issue(3, 0)
    scatter(1)

    @pl.loop(0, (NCHUNK - 5) // 3)
    def _triple(p):
        j = 3 * p + 2
        for k in range(3):
            b = (2 + k) % 3
            b2 = (4 + k) % 3
            drain_gather(b)
            compute(b)
            wait_scatter(b2)
            issue(j + k + 2, b2)
            scatter(b)

    _j = NCHUNK - 3
    drain_gather(_j % 3)
    compute(_j % 3)
    wait_scatter((_j + 2) % 3)
    issue(_j + 2, (_j + 2) % 3)
    scatter(_j % 3)
    for _j in range(NCHUNK - 2, NCHUNK):
        drain_gather(_j % 3)
        compute(_j % 3)
        scatter(_j % 3)
    for _b in range(3):
        wait_scatter(_b)

    plsc.subcore_barrier()
    pltpu.sync_copy(acc.at[pl.ds(s * ROWS_PER_TILE, ROWS_PER_TILE)],
                    out_hbm.at[c, pl.ds(s * ROWS_PER_TILE, ROWS_PER_TILE)])


def _combine_body(p_ref, msg_ref, scv_ref):
    p = p_ref[0] + p_ref[1]
    num = p[:, :HL]
    r0 = lax.broadcasted_iota(jnp.int32, (WIDTH, HL), 0)
    r1 = lax.broadcasted_iota(jnp.int32, (WIDTH, HL), 1)
    rep = ((r0 - 128) == (r1 // L)).astype(jnp.float32)
    den = jnp.dot(p, rep, preferred_element_type=jnp.float32,
                  precision=lax.Precision.HIGHEST)
    den = jnp.where(den != 0.0, den, 1.0)
    m = num / den
    m = jnp.where(m > 0.0, m, jnp.exp(m) - 1.0)
    msg_ref[...] = m
    scv_ref[...] = m[:, HL - 1:HL]


_combine = pl.pallas_call(
    _combine_body,
    grid=(20,),
    in_specs=[pl.BlockSpec((NC, 512, WIDTH), lambda i: (0, i, 0))],
    out_specs=[
        pl.BlockSpec((512, HL), lambda i: (i, 0)),
        pl.BlockSpec((512, 1), lambda i: (i, 0)),
    ],
    out_shape=[
        jax.ShapeDtypeStruct((NPAD, HL), jnp.float32),
        jax.ShapeDtypeStruct((NPAD, 1), jnp.float32),
    ],
)


def _head_body(msg_ref, sc_ref, c1w_ref, b1_ref, w2r_ref, b2_ref,
               out_ref, pooled_ref):
    nrows = NPAD // 128
    r_iota = lax.broadcasted_iota(jnp.int32, (nrows, 128), 0)
    l_iota = lax.broadcasted_iota(jnp.int32, (nrows, 128), 1)
    nid = r_iota * 128 + l_iota
    vals0 = jnp.where(nid < N, sc_ref[...], -jnp.inf)
    pooled_ref[...] = jnp.zeros((32, 128), jnp.float32)

    def body(i, vals):
        m = jnp.max(vals)
        idx = jnp.min(jnp.where(vals == m, nid, jnp.int32(2**30)))
        pooled_ref[pl.ds(i, 1), :] = msg_ref[pl.ds(idx, 1), :]
        return jnp.where(nid == idx, -jnp.inf, vals)

    lax.fori_loop(0, K, body, vals0)

    pooled = pooled_ref[...]
    x1t = lax.dot_general(c1w_ref[...], pooled, (((1,), (1,)), ((), ())),
                          preferred_element_type=jnp.float32)
    x1t = jnp.maximum(x1t + b1_ref[...], 0.0)

    j_i = lax.broadcasted_iota(jnp.int32, (32, K // 2), 0)
    p_i = lax.broadcasted_iota(jnp.int32, (32, K // 2), 1)
    se = (j_i == 2 * p_i).astype(jnp.float32)
    so = (j_i == 2 * p_i + 1).astype(jnp.float32)
    xp = jnp.maximum(
        jnp.dot(x1t, se, preferred_element_type=jnp.float32,
                precision=lax.Precision.HIGHEST),
        jnp.dot(x1t, so, preferred_element_type=jnp.float32,
                precision=lax.Precision.HIGHEST))

    npos = K // 2 - KW2 + 1
    y = jnp.zeros((C2, npos), jnp.float32)
    for t in range(KW2):
        w2t = w2r_ref[pl.ds(t * C2, C2), :]
        q_i = lax.broadcasted_iota(jnp.int32, (K // 2, npos), 0)
        s_i = lax.broadcasted_iota(jnp.int32, (K // 2, npos), 1)
        sel = (q_i == s_i + t).astype(jnp.float32)
        xpt = jnp.dot(xp, sel, preferred_element_type=jnp.float32,
                      precision=lax.Precision.HIGHEST)
        y = y + jnp.dot(w2t, xpt, preferred_element_type=jnp.float32)
    y = jnp.maximum(y + b2_ref[...], 0.0)
    out_ref[...] = jnp.concatenate(
        [y, jnp.zeros((C2, 128 - npos), jnp.float32)], axis=1)


_head = pl.pallas_call(
    _head_body,
    in_specs=[
        pl.BlockSpec(memory_space=pltpu.VMEM),
        pl.BlockSpec(memory_space=pltpu.VMEM),
        pl.BlockSpec(memory_space=pltpu.VMEM),
        pl.BlockSpec(memory_space=pltpu.VMEM),
        pl.BlockSpec(memory_space=pltpu.VMEM),
        pl.BlockSpec(memory_space=pltpu.VMEM),
    ],
    out_shape=jax.ShapeDtypeStruct((C2, 128), jnp.float32),
    scratch_shapes=[pltpu.VMEM((32, 128), jnp.float32)],
)


def kernel(node_feat, edge_index, W, a, conv1_w, conv1_b, conv2_w, conv2_b):
    f32 = jnp.float32
    wflat = W.transpose(1, 0, 2).reshape(D, HL)
    eye = jnp.eye(H, dtype=f32)
    a_src = a[:, :L]
    a_dst = a[:, L:]
    asrc_m = (a_src[:, :, None] * eye[:, None, :]).reshape(HL, H)
    adst_m = (a_dst[:, :, None] * eye[:, None, :]).reshape(HL, H)
    adst16 = jnp.concatenate([adst_m, jnp.zeros((HL, 12), f32)], axis=1)
    asrc16 = jnp.concatenate([asrc_m, jnp.zeros((HL, 12), f32)], axis=1)

    node_pad = jnp.pad(node_feat, ((0, NPAD - N), (0, 0)))

    hext, asrc = _proj(node_pad, wflat, adst16, asrc16)
    partial = _make_edge_kernel()(hext, asrc, edge_index)
    msg, scv = _combine(partial)
    sc2 = scv.reshape(NPAD // 128, 128)

    c1w = conv1_w[:, 0, :]
    b1 = conv1_b.reshape(C1, 1)
    w2r = conv2_w.transpose(2, 0, 1).reshape(KW2 * C2, C1)
    b2 = conv2_b.reshape(C2, 1)
    yfull = _head(msg, sc2, c1w, b1, w2r, b2)
    npos = K // 2 - KW2 + 1
    return yfull[:, :npos].reshape(1, C2 * npos)
